# baseline (speedup 1.0000x reference)
import functools

import jax
import jax.numpy as jnp
from jax import lax
from jax.experimental import pallas as pl
from jax.experimental.pallas import tpu as pltpu
from jax.experimental.pallas import tpu_sc as plsc

_PAD_CLS = 1024


def _fold_mlp_kernel(table_ref, w1_ref, b1_ref, w2_ref, b2_ref, out_ref):
    h = jnp.dot(table_ref[...], w1_ref[...], preferred_element_type=jnp.float32)
    h = h + b1_ref[...]
    h = h * jax.nn.sigmoid(h)
    out = jnp.dot(h, w2_ref[...], preferred_element_type=jnp.float32)
    out_ref[...] = out + b2_ref[...]


def _fold_mlp(table, W1, b1, W2, b2):
    n, d = table.shape
    return pl.pallas_call(
        _fold_mlp_kernel,
        grid=(1,),
        in_specs=[
            pl.BlockSpec((_PAD_CLS, d), lambda i: (0, 0)),
            pl.BlockSpec((d, d), lambda i: (0, 0)),
            pl.BlockSpec((1, d), lambda i: (0, 0)),
            pl.BlockSpec((d, d), lambda i: (0, 0)),
            pl.BlockSpec((1, d), lambda i: (0, 0)),
        ],
        out_specs=pl.BlockSpec((_PAD_CLS, d), lambda i: (0, 0)),
        out_shape=jax.ShapeDtypeStruct((_PAD_CLS, d), jnp.float32),
    )(table, W1, b1.reshape(1, d), W2, b2.reshape(1, d))


def _make_sc_gather(d, batch):
    info = plsc.get_sparse_core_info()
    nc, ns = info.num_cores, info.num_subcores
    nw = nc * ns
    b_per_w = batch // nw
    assert batch % (8 * nw) == 0
    chunk = min(128, b_per_w)
    n_chunks = b_per_w // chunk
    assert b_per_w % chunk == 0
    rows_per_tile = _PAD_CLS // ns

    mesh = plsc.VectorSubcoreMesh(core_axis_name="c", subcore_axis_name="s")
    rounds = 1
    cpr = n_chunks // rounds

    @functools.partial(
        pl.kernel,
        out_type=jax.ShapeDtypeStruct((batch, d), jnp.float32),
        mesh=mesh,
        scratch_types=[
            pltpu.VMEM((n_chunks, chunk), jnp.int32),
            pltpu.VMEM((cpr * chunk, d), jnp.float32),
            pltpu.VMEM_SHARED((_PAD_CLS, d), jnp.float32),
            [pltpu.SemaphoreType.DMA] * n_chunks,
            pltpu.SemaphoreType.DMA,
        ],
    )
    def gather(idx_hbm, tab_hbm, out_hbm, idx_v, rows_v, tab_sp, gsems, wsem):
        sid = lax.axis_index("s")
        wid = sid * nc + lax.axis_index("c")
        base = wid * b_per_w

        pltpu.sync_copy(idx_hbm.at[wid], idx_v)
        n_hbm = 2
        hbm_copies = [
            pltpu.async_copy(
                tab_hbm.at[idx_v.at[j]],
                rows_v.at[pl.ds(j * chunk, chunk)],
                gsems[j],
            )
            for j in range(n_hbm)
        ]
        tab_base = sid * rows_per_tile
        pltpu.sync_copy(
            tab_hbm.at[pl.ds(tab_base, rows_per_tile)],
            tab_sp.at[pl.ds(tab_base, rows_per_tile)],
        )
        plsc.subcore_barrier()

        for r in range(rounds):
            copies = []
            for k in range(cpr):
                j = r * cpr + k
                if j < n_hbm:
                    copies.append(hbm_copies[j])
                    continue
                copies.append(
                    pltpu.async_copy(
                        tab_sp.at[idx_v.at[j]],
                        rows_v.at[pl.ds(k * chunk, chunk)],
                        gsems[j],
                    )
                )
            writes = []
            for k in range(cpr):
                j = r * cpr + k
                copies[k].wait()
                writes.append(
                    pltpu.async_copy(
                        rows_v.at[pl.ds(k * chunk, chunk)],
                        out_hbm.at[pl.ds(base + j * chunk, chunk)],
                        wsem,
                    )
                )
            for w in writes:
                w.wait()

    def run(idx, tab):
        idx3 = idx.reshape(nw, n_chunks, chunk)
        return gather(idx3, tab)

    return run


def kernel(x, table, W1, b1, W2, b2):
    batch = x.shape[0]
    n, d = table.shape
    t2 = _fold_mlp(table, W1, b1, W2, b2)
    gather = _make_sc_gather(d, batch)
    return gather(x.astype(jnp.int32), t2)


# speedup vs baseline: 1.0836x; 1.0836x over previous
import functools

import jax
import jax.numpy as jnp
from jax import lax
from jax.experimental import pallas as pl
from jax.experimental.pallas import tpu as pltpu
from jax.experimental.pallas import tpu_sc as plsc

_PAD_CLS = 1024


def _fold_mlp_kernel(table_ref, w1_ref, b1_ref, w2_ref, b2_ref, out_ref):
    h = jnp.dot(table_ref[...], w1_ref[...], preferred_element_type=jnp.float32)
    h = h + b1_ref[...]
    h = h * jax.nn.sigmoid(h)
    out = jnp.dot(h, w2_ref[...], preferred_element_type=jnp.float32)
    out_ref[...] = out + b2_ref[...]


def _fold_mlp(table, W1, b1, W2, b2):
    n, d = table.shape
    return pl.pallas_call(
        _fold_mlp_kernel,
        grid=(1,),
        in_specs=[
            pl.BlockSpec((_PAD_CLS, d), lambda i: (0, 0)),
            pl.BlockSpec((d, d), lambda i: (0, 0)),
            pl.BlockSpec((1, d), lambda i: (0, 0)),
            pl.BlockSpec((d, d), lambda i: (0, 0)),
            pl.BlockSpec((1, d), lambda i: (0, 0)),
        ],
        out_specs=pl.BlockSpec((_PAD_CLS, d), lambda i: (0, 0)),
        out_shape=jax.ShapeDtypeStruct((_PAD_CLS, d), jnp.float32),
    )(table, W1, b1.reshape(1, d), W2, b2.reshape(1, d))


def _make_sc_gather(d, batch):
    info = plsc.get_sparse_core_info()
    nc, ns = info.num_cores, info.num_subcores
    nw = nc * ns
    b_per_w = batch // nw
    assert batch % (8 * nw) == 0
    chunk = min(128, b_per_w)
    n_chunks = b_per_w // chunk
    assert b_per_w % chunk == 0
    rows_per_tile = _PAD_CLS // ns

    mesh = plsc.VectorSubcoreMesh(core_axis_name="c", subcore_axis_name="s")
    rounds = 1
    cpr = n_chunks // rounds

    @functools.partial(
        pl.kernel,
        out_type=jax.ShapeDtypeStruct((batch, d), jnp.float32),
        mesh=mesh,
        scratch_types=[
            pltpu.VMEM((n_chunks, chunk), jnp.int32),
            pltpu.VMEM((cpr * chunk, d), jnp.float32),
            pltpu.VMEM_SHARED((_PAD_CLS, d), jnp.float32),
            [pltpu.SemaphoreType.DMA] * n_chunks,
            pltpu.SemaphoreType.DMA,
        ],
    )
    def gather(idx_hbm, tab_hbm, out_hbm, idx_v, rows_v, tab_sp, gsems, wsem):
        sid = lax.axis_index("s")
        wid = sid * nc + lax.axis_index("c")
        base = wid * b_per_w

        pltpu.sync_copy(idx_hbm.at[wid], idx_v)
        n_hbm = 1
        hbm_copies = [
            pltpu.async_copy(
                tab_hbm.at[idx_v.at[j]],
                rows_v.at[pl.ds(j * chunk, chunk)],
                gsems[j],
            )
            for j in range(n_hbm)
        ]
        tab_base = sid * rows_per_tile
        pltpu.sync_copy(
            tab_hbm.at[pl.ds(tab_base, rows_per_tile)],
            tab_sp.at[pl.ds(tab_base, rows_per_tile)],
        )
        plsc.subcore_barrier()

        for r in range(rounds):
            copies = []
            for k in range(cpr):
                j = r * cpr + k
                if j < n_hbm:
                    copies.append(hbm_copies[j])
                    continue
                copies.append(
                    pltpu.async_copy(
                        tab_sp.at[idx_v.at[j]],
                        rows_v.at[pl.ds(k * chunk, chunk)],
                        gsems[j],
                    )
                )
            writes = []
            for k in range(cpr):
                j = r * cpr + k
                copies[k].wait()
                writes.append(
                    pltpu.async_copy(
                        rows_v.at[pl.ds(k * chunk, chunk)],
                        out_hbm.at[pl.ds(base + j * chunk, chunk)],
                        wsem,
                    )
                )
            for w in writes:
                w.wait()

    def run(idx, tab):
        idx3 = idx.reshape(nw, n_chunks, chunk)
        return gather(idx3, tab)

    return run


def kernel(x, table, W1, b1, W2, b2):
    batch = x.shape[0]
    n, d = table.shape
    t2 = _fold_mlp(table, W1, b1, W2, b2)
    gather = _make_sc_gather(d, batch)
    return gather(x.astype(jnp.int32), t2)
